# Initial kernel scaffold; baseline (speedup 1.0000x reference)
import jax
import jax.numpy as jnp
from jax.experimental import pallas as pl


def kernel(node_types, node_trust, type_table, trust_table):
    raise NotImplementedError("write your pallas kernel here")


# speedup vs baseline: 1.2947x; 1.2947x over previous
import functools

import jax
import jax.numpy as jnp
from jax import lax
from jax.experimental import pallas as pl
from jax.experimental.pallas import tpu as pltpu
from jax.experimental.pallas import tpu_sc as plsc

VOCAB = 1000000
NUM_TRUST = 6
DIM = 64
LANES = 16
NUM_CORES = 2
NUM_SUBCORES = 16
NW = NUM_CORES * NUM_SUBCORES

CHUNK = 1024
SUB = 128


def _embed_body(types_hbm, trusts_hbm, type_table_hbm, trust_flat_hbm,
                out_hbm, idx_v, tidx_v, rows_v, trust_v, gsem):
    n = out_hbm.shape[0]
    per_w = n // NW
    nchunks = per_w // CHUNK
    wid = lax.axis_index("s") * NUM_CORES + lax.axis_index("c")

    pltpu.sync_copy(trust_flat_hbm, trust_v)

    iota16 = lax.iota(jnp.int32, LANES)

    def chunk_body(k, _):
        base = pl.multiple_of(wid * per_w + k * CHUNK, CHUNK)

        pltpu.sync_copy(types_hbm.at[pl.ds(base, CHUNK)], idx_v)
        pltpu.sync_copy(trusts_hbm.at[pl.ds(base, CHUNK)], tidx_v)

        copies = []
        for j in range(CHUNK // SUB):
            c = pltpu.make_async_copy(
                type_table_hbm.at[idx_v.at[pl.ds(j * SUB, SUB)]],
                rows_v.at[pl.ds(j * SUB, SUB)],
                gsem,
            )
            c.start()
            copies.append(c)
        for c in copies:
            c.wait()

        def group_body(g, _):
            t_vec = tidx_v[pl.ds(g * LANES, LANES)]
            tbase = t_vec * DIM
            row_vec = g * LANES + iota16
            for d in range(DIM):
                dvec = jnp.full((LANES,), d, jnp.int32)
                tvals = plsc.load_gather(trust_v, [tbase + d])
                plsc.addupdate_scatter(rows_v, [row_vec, dvec], tvals)
            return 0

        lax.fori_loop(0, CHUNK // LANES, group_body, 0, unroll=False)

        pltpu.sync_copy(rows_v, out_hbm.at[pl.ds(base, CHUNK)])
        return 0

    lax.fori_loop(0, nchunks, chunk_body, 0, unroll=False)


@jax.jit
def kernel(node_types, node_trust, type_table, trust_table):
    b, l = node_types.shape
    n = b * l
    types_flat = node_types.reshape(n).astype(jnp.int32)
    trusts_flat = node_trust.reshape(n).astype(jnp.int32)
    trust_flat = trust_table.reshape(NUM_TRUST * DIM)

    mesh = plsc.VectorSubcoreMesh(
        core_axis_name="c", subcore_axis_name="s",
        num_cores=NUM_CORES, num_subcores=NUM_SUBCORES)

    run = pl.kernel(
        _embed_body,
        out_type=jax.ShapeDtypeStruct((n, DIM), jnp.float32),
        mesh=mesh,
        compiler_params=pltpu.CompilerParams(
            needs_layout_passes=False, use_tc_tiling_on_sc=False),
        scratch_types=[
            pltpu.VMEM((CHUNK,), jnp.int32),
            pltpu.VMEM((CHUNK,), jnp.int32),
            pltpu.VMEM((CHUNK, DIM), jnp.float32),
            pltpu.VMEM((NUM_TRUST * DIM,), jnp.float32),
            pltpu.SemaphoreType.DMA,
        ],
    )
    out = run(types_flat, trusts_flat, type_table, trust_flat)
    return out.reshape(b, l, DIM)


# speedup vs baseline: 3.4886x; 2.6945x over previous
import jax
import jax.numpy as jnp
from jax import lax
from jax.experimental import pallas as pl
from jax.experimental.pallas import tpu as pltpu
from jax.experimental.pallas import tpu_sc as plsc

NUM_TRUST = 6
DIM = 64
LANES = 16
NUM_CORES = 2
NUM_SUBCORES = 16
NW = NUM_CORES * NUM_SUBCORES

CHUNK = 512
SUB = 128
NBUF = 3


def _embed_body(types_hbm, trusts_hbm, type_table_hbm, trust_flat_hbm,
                out_hbm,
                i0, i1, i2, t0, t1, t2, r0, r1, r2, trust_v,
                g0, g1, g2, o0, o1, o2):
    ibufs = (i0, i1, i2)
    tbufs = (t0, t1, t2)
    rows = (r0, r1, r2)
    gsems = (g0, g1, g2)
    osems = (o0, o1, o2)

    n = out_hbm.shape[0]
    per_w = n // NW
    nch = per_w // CHUNK
    assert per_w % CHUNK == 0 and (nch - 2) % NBUF == 0, (per_w, nch)
    wid = lax.axis_index("s") * NUM_CORES + lax.axis_index("c")
    wbase = wid * per_w

    pltpu.sync_copy(trust_flat_hbm, trust_v)

    def gather_copies(k, r):
        cs = []
        for j in range(CHUNK // SUB):
            cs.append(pltpu.make_async_copy(
                type_table_hbm.at[ibufs[r].at[pl.ds(j * SUB, SUB)]],
                rows[r].at[pl.ds(j * SUB, SUB)],
                gsems[r]))
        return cs

    def out_copy(k, r):
        base = pl.multiple_of(wbase + k * CHUNK, CHUNK)
        return pltpu.make_async_copy(rows[r], out_hbm.at[pl.ds(base, CHUNK)],
                                     osems[r])

    def start(k, r, wait_out):
        base = pl.multiple_of(wbase + k * CHUNK, CHUNK)
        pltpu.sync_copy(types_hbm.at[pl.ds(base, CHUNK)], ibufs[r])
        pltpu.sync_copy(trusts_hbm.at[pl.ds(base, CHUNK)], tbufs[r])
        if wait_out == "always":
            out_copy(k - NBUF, r).wait()
        elif wait_out == "guarded":
            @pl.when(k >= NBUF)
            def _():
                out_copy(k - NBUF, r).wait()
        for c in gather_copies(k, r):
            c.start()

    def finish(k, r):
        for c in gather_copies(k, r):
            c.wait()

        def group_body(g, _):
            t_vec = tbufs[r][pl.ds(g * LANES, LANES)] * DIM
            for lane in range(LANES):
                toff = t_vec[lane]
                i = g * LANES + lane
                for db in range(DIM // LANES):
                    plsc.addupdate(
                        rows[r].at[i, pl.ds(db * LANES, LANES)],
                        trust_v[pl.ds(toff + db * LANES, LANES)])
            return 0

        lax.fori_loop(0, CHUNK // LANES, group_body, 0, unroll=False)
        out_copy(k, r).start()

    start(0, 0, wait_out="none")
    start(1, 1, wait_out="none")

    nsup = (nch - 2) // NBUF

    def super_body(s, _):
        for j in range(NBUF):
            k = s * NBUF + j
            ks = k + 2
            rs = (j + 2) % NBUF
            start(ks, rs, wait_out="guarded" if j == 0 else "always")
            finish(k, j)
        return 0

    lax.fori_loop(0, nsup, super_body, 0, unroll=False)

    for k in range(nsup * NBUF, nch):
        finish(k, k % NBUF)

    for k in range(max(0, nch - NBUF), nch):
        out_copy(k, k % NBUF).wait()


@jax.jit
def kernel(node_types, node_trust, type_table, trust_table):
    b, l = node_types.shape
    n = b * l
    types_flat = node_types.reshape(n).astype(jnp.int32)
    trusts_flat = node_trust.reshape(n).astype(jnp.int32)
    trust_flat = trust_table.reshape(NUM_TRUST * DIM)

    mesh = plsc.VectorSubcoreMesh(
        core_axis_name="c", subcore_axis_name="s",
        num_cores=NUM_CORES, num_subcores=NUM_SUBCORES)

    run = pl.kernel(
        _embed_body,
        out_type=jax.ShapeDtypeStruct((n, DIM), jnp.float32),
        mesh=mesh,
        compiler_params=pltpu.CompilerParams(
            needs_layout_passes=False, use_tc_tiling_on_sc=False),
        scratch_types=(
            [pltpu.VMEM((CHUNK,), jnp.int32) for _ in range(NBUF)]
            + [pltpu.VMEM((CHUNK,), jnp.int32) for _ in range(NBUF)]
            + [pltpu.VMEM((CHUNK, DIM), jnp.float32) for _ in range(NBUF)]
            + [pltpu.VMEM((NUM_TRUST * DIM,), jnp.float32)]
            + [pltpu.SemaphoreType.DMA for _ in range(2 * NBUF)]
        ),
    )
    out = run(types_flat, trusts_flat, type_table, trust_flat)
    return out.reshape(b, l, DIM)
